# baseline (speedup 1.0000x reference)
import functools
import jax
import jax.numpy as jnp
from jax import lax
from jax.experimental import pallas as pl
from jax.experimental.pallas import tpu as pltpu
from jax.experimental.pallas import tpu_sc as plsc

NC = 2
NS = 16
L = 16
NW = NC * NS


def _copy_body(mem_ref, out_ref):
    out_ref[...] = mem_ref[...]


def kernel(memory, node_idxs, values):
    N, D = memory.shape
    B = node_idxs.shape[0]
    SPAN = N // NW
    NPAIRS = B // (2 * L)
    WSLOTS = ((SPAN + L - 1) // L) * L
    NWG = WSLOTS // L
    C = 128
    COMP_SZ = WSLOTS + C + L

    mesh = plsc.VectorSubcoreMesh(
        core_axis_name="c", subcore_axis_name="s",
        num_cores=NC, num_subcores=NS)
    sc_params = pltpu.CompilerParams(
        use_tc_tiling_on_sc=False, needs_layout_passes=False)

    @functools.partial(
        pl.kernel,
        out_type=(jax.ShapeDtypeStruct((NW, COMP_SZ), jnp.int32),
                  jax.ShapeDtypeStruct((NW, L), jnp.int32)),
        mesh=mesh,
        compiler_params=sc_params,
        scratch_types=[
            pltpu.VMEM((B,), jnp.int32),
            pltpu.VMEM((WSLOTS,), jnp.int32),
            pltpu.VMEM((COMP_SZ,), jnp.int32),
            pltpu.VMEM((L,), jnp.int32),
            pltpu.SemaphoreType.DMA,
        ],
    )
    def sc_dedup(idx_hbm, comp_hbm, meta_hbm,
                 idx_v, winner, comp, meta_v, xsem):
        wid = lax.axis_index("s") * NC + lax.axis_index("c")
        base_n = wid * SPAN

        xdesc = pltpu.async_copy(idx_hbm, idx_v, xsem)

        iota = lax.iota(jnp.int32, L)
        nxt_perm = jnp.minimum(iota + 1, L - 1)
        neg1 = jnp.full((L,), -1, jnp.int32)

        def init_body(k, carry):
            winner[pl.ds(k * L, L)] = neg1
            return carry
        lax.fori_loop(0, NWG, init_body, 0)

        xdesc.wait()

        def dedup_group(nodes, pos_base):
            pval = lax.shift_left(nodes, 14) | (pos_base + iota)
            spval = lax.sort(pval)
            snode = lax.shift_right_logical(spval, 14)
            nxt = snode.at[nxt_perm].get(mode="promise_in_bounds")
            sd = snode - base_n
            m = ((snode != nxt) | (iota == L - 1)) \
                & (plsc.bitcast(sd, jnp.uint32) < jnp.uint32(SPAN))
            slot = jnp.where(m, sd, 0)
            plsc.store_scatter(winner, [slot], spval, mask=m)

        def scan_body(gg, carry):
            nodes0 = idx_v[pl.ds(gg * 2 * L, L)]
            nodes1 = idx_v[pl.ds(gg * 2 * L + L, L)]
            inr0 = plsc.bitcast(nodes0 - base_n, jnp.uint32) < jnp.uint32(SPAN)
            inr1 = plsc.bitcast(nodes1 - base_n, jnp.uint32) < jnp.uint32(SPAN)

            @pl.when(jnp.any(inr0 | inr1))
            def _():
                @pl.when(jnp.any(inr0))
                def _():
                    dedup_group(nodes0, gg * 2 * L)
                @pl.when(jnp.any(inr1))
                def _():
                    dedup_group(nodes1, gg * 2 * L + L)
            return carry
        lax.fori_loop(0, NPAIRS, scan_body, 0)

        def comp_body(k, carry):
            off, lastv = carry
            w = winner[pl.ds(k * L, L)]
            m = w >= 0
            incl = plsc.cumsum(m.astype(jnp.int32))
            cnt = jnp.max(incl)
            tgt = jnp.where(m, off + incl - 1, 0)
            plsc.store_scatter(comp, [tgt], w, mask=m)
            lastv = jnp.maximum(lastv, jnp.max(jnp.where(m, w, -1)))
            return off + cnt, lastv
        n_sel, lastv = lax.fori_loop(
            0, NWG, comp_body, (jnp.int32(0), jnp.int32(-1)))

        lastv_v = jnp.broadcast_to(lastv, (L,))
        def pad_body(t, carry):
            plsc.store_scatter(comp, [n_sel + t * L + iota], lastv_v)
            return carry
        lax.fori_loop(0, C // L, pad_body, 0)

        meta_v[pl.ds(0, L)] = jnp.broadcast_to(n_sel, (L,))
        pltpu.sync_copy(comp, comp_hbm.at[wid])
        pltpu.sync_copy(meta_v, meta_hbm.at[wid])

    CB = 10000
    tc_out = pl.pallas_call(
        _copy_body,
        out_shape=jax.ShapeDtypeStruct((N, D), jnp.float32),
        grid=(N // CB,),
        in_specs=[pl.BlockSpec((CB, D), lambda i: (i, 0))],
        out_specs=pl.BlockSpec((CB, D), lambda i: (i, 0)),
    )(memory)

    @functools.partial(
        pl.kernel,
        mesh=mesh,
        compiler_params=sc_params,
        scratch_types=[
            pltpu.VMEM((COMP_SZ,), jnp.int32),
            pltpu.VMEM((L,), jnp.int32),
            pltpu.VMEM((2, C), jnp.int32),
            pltpu.VMEM((2, C), jnp.int32),
            pltpu.VMEM((2, C, 128), jnp.float32),
            pltpu.SemaphoreType.DMA,
            pltpu.SemaphoreType.DMA,
            pltpu.SemaphoreType.DMA,
        ],
    )
    def sc_scatter(comp_hbm, meta_hbm, val_hbm, out_hbm,
                   comp, meta_v, idxrow, posrow, vbuf, xsem, gsem, ssem):
        wid = lax.axis_index("s") * NC + lax.axis_index("c")
        cdesc = pltpu.async_copy(comp_hbm.at[wid], comp, xsem)
        pltpu.sync_copy(meta_hbm.at[wid], meta_v)
        n_sel = jnp.max(meta_v[pl.ds(0, L)])
        n_chunks = (n_sel + C - 1) // C
        cdesc.wait()

        def build_rows(j):
            s = j % 2
            def b_body(t, carry):
                p = comp[pl.ds(j * C + t * L, L)]
                idxrow[s, pl.ds(t * L, L)] = lax.shift_right_logical(p, 14)
                posrow[s, pl.ds(t * L, L)] = lax.bitwise_and(p, 16383)
                return carry
            lax.fori_loop(0, C // L, b_body, 0)

        @pl.when(n_chunks > 0)
        def _():
            build_rows(jnp.int32(0))
            pltpu.async_copy(val_hbm.at[posrow.at[0]], vbuf.at[0], gsem)

        def chunk_body(j, carry):
            s = j % 2
            pltpu.make_async_copy(
                val_hbm.at[posrow.at[s]], vbuf.at[s], gsem).wait()
            pltpu.async_copy(vbuf.at[s], out_hbm.at[idxrow.at[s]], ssem)
            @pl.when(j + 1 < n_chunks)
            def _():
                build_rows(j + 1)
                @pl.when(j >= 1)
                def _():
                    pltpu.make_async_copy(
                        vbuf.at[1 - s], out_hbm.at[idxrow.at[1 - s]],
                        ssem).wait()
                pltpu.async_copy(
                    val_hbm.at[posrow.at[1 - s]], vbuf.at[1 - s], gsem)
            return carry
        lax.fori_loop(0, n_chunks, chunk_body, 0)

        @pl.when(n_chunks >= 2)
        def _():
            pltpu.make_async_copy(
                vbuf.at[0], out_hbm.at[idxrow.at[0]], ssem).wait()
        @pl.when(n_chunks >= 1)
        def _():
            pltpu.make_async_copy(
                vbuf.at[0], out_hbm.at[idxrow.at[0]], ssem).wait()

    comp_all, meta_all = sc_dedup(node_idxs)
    out_ref = jax.new_ref(tc_out)
    sc_scatter(comp_all, meta_all, values, out_ref)
    return jax.freeze(out_ref)


# speedup vs baseline: 1.0490x; 1.0490x over previous
import functools
import jax
import jax.numpy as jnp
from jax import lax
from jax.experimental import pallas as pl
from jax.experimental.pallas import tpu as pltpu
from jax.experimental.pallas import tpu_sc as plsc

NC = 2
NS = 16
L = 16
NW = NC * NS


def _copy_body(mem_ref, out_ref):
    out_ref[...] = mem_ref[...]


def kernel(memory, node_idxs, values):
    N, D = memory.shape
    B = node_idxs.shape[0]
    SPAN = N // NW
    NPAIRS = B // (2 * L)
    WSLOTS = ((SPAN + L - 1) // L) * L
    NWG = WSLOTS // L
    C = 128
    COMP_SZ = WSLOTS + C + L

    mesh = plsc.VectorSubcoreMesh(
        core_axis_name="c", subcore_axis_name="s",
        num_cores=NC, num_subcores=NS)
    sc_params = pltpu.CompilerParams(
        use_tc_tiling_on_sc=False, needs_layout_passes=False)

    @functools.partial(
        pl.kernel,
        out_type=(jax.ShapeDtypeStruct((NW, COMP_SZ), jnp.int32),
                  jax.ShapeDtypeStruct((NW, L), jnp.int32)),
        mesh=mesh,
        compiler_params=sc_params,
        scratch_types=[
            pltpu.VMEM((B,), jnp.int32),
            pltpu.VMEM((WSLOTS,), jnp.int32),
            pltpu.VMEM((COMP_SZ,), jnp.int32),
            pltpu.VMEM((L,), jnp.int32),
            pltpu.SemaphoreType.DMA,
        ],
    )
    def sc_dedup(idx_hbm, comp_hbm, meta_hbm,
                 idx_v, winner, comp, meta_v, xsem):
        wid = lax.axis_index("s") * NC + lax.axis_index("c")
        base_n = wid * SPAN

        xdesc = pltpu.async_copy(idx_hbm, idx_v, xsem)

        iota = lax.iota(jnp.int32, L)
        nxt_perm = jnp.minimum(iota + 1, L - 1)
        neg1 = jnp.full((L,), -1, jnp.int32)

        def init_body(k, carry):
            winner[pl.ds(k * L, L)] = neg1
            return carry
        lax.fori_loop(0, NWG, init_body, 0)

        xdesc.wait()

        def dedup_group(nodes, pos_base):
            pval = lax.shift_left(nodes, 14) | (pos_base + iota)
            spval = lax.sort(pval)
            snode = lax.shift_right_logical(spval, 14)
            nxt = snode.at[nxt_perm].get(mode="promise_in_bounds")
            sd = snode - base_n
            m = ((snode != nxt) | (iota == L - 1)) \
                & (plsc.bitcast(sd, jnp.uint32) < jnp.uint32(SPAN))
            slot = jnp.where(m, sd, 0)
            plsc.store_scatter(winner, [slot], spval, mask=m)

        def scan_body(gg, carry):
            nodes0 = idx_v[pl.ds(gg * 2 * L, L)]
            nodes1 = idx_v[pl.ds(gg * 2 * L + L, L)]
            inr0 = plsc.bitcast(nodes0 - base_n, jnp.uint32) < jnp.uint32(SPAN)
            inr1 = plsc.bitcast(nodes1 - base_n, jnp.uint32) < jnp.uint32(SPAN)

            @pl.when(jnp.any(inr0 | inr1))
            def _():
                @pl.when(jnp.any(inr0))
                def _():
                    dedup_group(nodes0, gg * 2 * L)
                @pl.when(jnp.any(inr1))
                def _():
                    dedup_group(nodes1, gg * 2 * L + L)
            return carry
        lax.fori_loop(0, NPAIRS, scan_body, 0)

        def comp_body(k, carry):
            off, lastv = carry
            w = winner[pl.ds(k * L, L)]
            m = w >= 0
            incl = plsc.cumsum(m.astype(jnp.int32))
            cnt = jnp.max(incl)
            tgt = jnp.where(m, off + incl - 1, 0)
            plsc.store_scatter(comp, [tgt], w, mask=m)
            lastv = jnp.maximum(lastv, jnp.max(jnp.where(m, w, -1)))
            return off + cnt, lastv
        n_sel, lastv = lax.fori_loop(
            0, NWG, comp_body, (jnp.int32(0), jnp.int32(-1)))

        lastv_v = jnp.broadcast_to(lastv, (L,))
        def pad_body(t, carry):
            plsc.store_scatter(comp, [n_sel + t * L + iota], lastv_v)
            return carry
        lax.fori_loop(0, C // L, pad_body, 0)

        meta_v[pl.ds(0, L)] = jnp.broadcast_to(n_sel, (L,))
        pltpu.sync_copy(comp, comp_hbm.at[wid])
        pltpu.sync_copy(meta_v, meta_hbm.at[wid])

    CB = 4000
    tc_out = pl.pallas_call(
        _copy_body,
        out_shape=jax.ShapeDtypeStruct((N, D), jnp.float32),
        grid=(N // CB,),
        in_specs=[pl.BlockSpec((CB, D), lambda i: (i, 0))],
        out_specs=pl.BlockSpec((CB, D), lambda i: (i, 0)),
    )(memory)

    @functools.partial(
        pl.kernel,
        mesh=mesh,
        compiler_params=sc_params,
        scratch_types=[
            pltpu.VMEM((COMP_SZ,), jnp.int32),
            pltpu.VMEM((L,), jnp.int32),
            pltpu.VMEM((2, C), jnp.int32),
            pltpu.VMEM((2, C), jnp.int32),
            pltpu.VMEM((2, C, 128), jnp.float32),
            pltpu.SemaphoreType.DMA,
            pltpu.SemaphoreType.DMA,
            pltpu.SemaphoreType.DMA,
        ],
    )
    def sc_scatter(comp_hbm, meta_hbm, val_hbm, out_hbm,
                   comp, meta_v, idxrow, posrow, vbuf, xsem, gsem, ssem):
        wid = lax.axis_index("s") * NC + lax.axis_index("c")
        cdesc = pltpu.async_copy(comp_hbm.at[wid], comp, xsem)
        pltpu.sync_copy(meta_hbm.at[wid], meta_v)
        n_sel = jnp.max(meta_v[pl.ds(0, L)])
        n_chunks = (n_sel + C - 1) // C
        cdesc.wait()

        def build_rows(j):
            s = j % 2
            def b_body(t, carry):
                p = comp[pl.ds(j * C + t * L, L)]
                idxrow[s, pl.ds(t * L, L)] = lax.shift_right_logical(p, 14)
                posrow[s, pl.ds(t * L, L)] = lax.bitwise_and(p, 16383)
                return carry
            lax.fori_loop(0, C // L, b_body, 0)

        @pl.when(n_chunks > 0)
        def _():
            build_rows(jnp.int32(0))
            pltpu.async_copy(val_hbm.at[posrow.at[0]], vbuf.at[0], gsem)

        def chunk_body(j, carry):
            s = j % 2
            pltpu.make_async_copy(
                val_hbm.at[posrow.at[s]], vbuf.at[s], gsem).wait()
            pltpu.async_copy(vbuf.at[s], out_hbm.at[idxrow.at[s]], ssem)
            @pl.when(j + 1 < n_chunks)
            def _():
                build_rows(j + 1)
                @pl.when(j >= 1)
                def _():
                    pltpu.make_async_copy(
                        vbuf.at[1 - s], out_hbm.at[idxrow.at[1 - s]],
                        ssem).wait()
                pltpu.async_copy(
                    val_hbm.at[posrow.at[1 - s]], vbuf.at[1 - s], gsem)
            return carry
        lax.fori_loop(0, n_chunks, chunk_body, 0)

        @pl.when(n_chunks >= 2)
        def _():
            pltpu.make_async_copy(
                vbuf.at[0], out_hbm.at[idxrow.at[0]], ssem).wait()
        @pl.when(n_chunks >= 1)
        def _():
            pltpu.make_async_copy(
                vbuf.at[0], out_hbm.at[idxrow.at[0]], ssem).wait()

    comp_all, meta_all = sc_dedup(node_idxs)
    out_ref = jax.new_ref(tc_out)
    sc_scatter(comp_all, meta_all, values, out_ref)
    return jax.freeze(out_ref)
